# Initial kernel scaffold; baseline (speedup 1.0000x reference)
import jax
import jax.numpy as jnp
from jax.experimental import pallas as pl


def kernel(x, edge_index, params):
    raise NotImplementedError("write your pallas kernel here")


# speedup vs baseline: 1.0029x; 1.0029x over previous
import functools
import math

import numpy as np
import jax
import jax.numpy as jnp
from jax import lax
from jax.experimental import pallas as pl
from jax.experimental.pallas import tpu as pltpu

HID = 8
PACK = 16
LANES = PACK * HID


def _pick_rows_blk(rows):
    for cand in (2048, 2000, 1600, 1280, 1024, 1000, 800, 640, 512, 400,
                 320, 256, 200, 160, 128, 80, 64, 40, 32, 16, 8):
        if rows % cand == 0:
            return cand
    return rows


def _kron16(w):
    return jnp.kron(jnp.eye(PACK, dtype=jnp.float32), w.astype(jnp.float32))


def _tile16(v):
    return jnp.tile(v.astype(jnp.float32).reshape(-1), PACK).reshape(1, LANES)


def _ln_tanh(h, dm, g, b):
    m = jnp.dot(h, dm)
    c = h - m
    v = jnp.dot(c * c, dm)
    return jnp.tanh(c * lax.rsqrt(v + 1e-5) * g + b)


def _edge_body(xs_ref, xe_ref, dm_ref, d1a_ref, d1b_ref, d2_ref, d3_ref,
               d4_ref, db_ref, v_ref, ws_ref, we_ref):
    xs = xs_ref[...]
    xe = xe_ref[...]
    dm = dm_ref[...]
    v = v_ref[...]
    h = jnp.dot(xs, d1a_ref[...]) + jnp.dot(xe, d1b_ref[...]) + v[0:1]
    h = _ln_tanh(h, dm, v[1:2], v[2:3])
    h = jnp.dot(h, d2_ref[...]) + v[3:4]
    h = _ln_tanh(h, dm, v[4:5], v[5:6])
    h = jnp.dot(h, d3_ref[...]) + v[6:7]
    h = _ln_tanh(h, dm, v[7:8], v[8:9])
    lg = jnp.dot(h, d4_ref[...]) + v[9:10]
    e = jnp.dot(jax.nn.sigmoid(lg), db_ref[...])
    ws_ref[...] = e * xs
    we_ref[...] = e * xe


def _edge_final_body(xs_ref, xe_ref, dm_ref, d1a_ref, d1b_ref, d2_ref,
                     d3_ref, d4_ref, dc_ref, v_ref, out_ref):
    xs = xs_ref[...]
    xe = xe_ref[...]
    dm = dm_ref[...]
    v = v_ref[...]
    h = jnp.dot(xs, d1a_ref[...]) + jnp.dot(xe, d1b_ref[...]) + v[0:1]
    h = _ln_tanh(h, dm, v[1:2], v[2:3])
    h = jnp.dot(h, d2_ref[...]) + v[3:4]
    h = _ln_tanh(h, dm, v[4:5], v[5:6])
    h = jnp.dot(h, d3_ref[...]) + v[6:7]
    h = _ln_tanh(h, dm, v[7:8], v[8:9])
    lg = jnp.dot(h, d4_ref[...]) + v[9:10]
    out_ref[...] = jnp.dot(lg, dc_ref[...])


def _node_body(mi_ref, mo_ref, x_ref, dm_ref, da_ref, dbm_ref, dc_ref,
               d2_ref, d3_ref, d4_ref, v_ref, out_ref):
    x = x_ref[...]
    dm = dm_ref[...]
    v = v_ref[...]
    h = (jnp.dot(mi_ref[...], da_ref[...]) + jnp.dot(mo_ref[...], dbm_ref[...])
         + jnp.dot(x, dc_ref[...]) + v[0:1])
    h = _ln_tanh(h, dm, v[1:2], v[2:3])
    h = jnp.dot(h, d2_ref[...]) + v[3:4]
    h = _ln_tanh(h, dm, v[4:5], v[5:6])
    h = jnp.dot(h, d3_ref[...]) + v[6:7]
    h = _ln_tanh(h, dm, v[7:8], v[8:9])
    h = jnp.dot(h, d4_ref[...]) + v[9:10]
    h = _ln_tanh(h, dm, v[10:11], v[11:12])
    out_ref[...] = h + x


def _inp_body(x_ref, dm_ref, d1_ref, v_ref, out_ref):
    dm = dm_ref[...]
    v = v_ref[...]
    h = jnp.dot(x_ref[...], d1_ref[...]) + v[0:1]
    out_ref[...] = _ln_tanh(h, dm, v[1:2], v[2:3])


def _rep_spec(shape):
    nd = len(shape)
    zero = np.int32(0)
    return pl.BlockSpec(shape, lambda *_: (zero,) * nd)


def _edge_pass(xs_p, xe_p, mats, vecs, final):
    rows = xs_p.shape[0]
    blk = _pick_rows_blk(rows)
    grid = rows // blk
    row_spec = pl.BlockSpec((blk, LANES), lambda i: (i, np.int32(0)))
    in_specs = [row_spec, row_spec] + [_rep_spec(m.shape) for m in mats] + \
               [_rep_spec(vecs.shape)]
    if final:
        out_specs = pl.BlockSpec((blk, PACK), lambda i: (i, np.int32(0)))
        out_shape = jax.ShapeDtypeStruct((rows, PACK), jnp.float32)
        body = _edge_final_body
    else:
        out_specs = (row_spec, row_spec)
        out_shape = (jax.ShapeDtypeStruct((rows, LANES), jnp.float32),
                     jax.ShapeDtypeStruct((rows, LANES), jnp.float32))
        body = _edge_body
    return pl.pallas_call(
        body, grid=(grid,), in_specs=in_specs, out_specs=out_specs,
        out_shape=out_shape)(xs_p, xe_p, *mats, vecs)


def _node_pass(mi_p, mo_p, x_p, mats, vecs):
    rows = mi_p.shape[0]
    specs = [_rep_spec(a.shape) for a in (mi_p, mo_p, x_p)] + \
            [_rep_spec(m.shape) for m in mats] + [_rep_spec(vecs.shape)]
    return pl.pallas_call(
        _node_body, in_specs=specs, out_specs=_rep_spec(x_p.shape),
        out_shape=jax.ShapeDtypeStruct(x_p.shape, jnp.float32),
    )(mi_p, mo_p, x_p, *mats, vecs)


def _inp_pass(x_p, mats, vecs):
    specs = [_rep_spec(x_p.shape)] + [_rep_spec(m.shape) for m in mats] + \
            [_rep_spec(vecs.shape)]
    return pl.pallas_call(
        _inp_body, in_specs=specs, out_specs=_rep_spec(x_p.shape),
        out_shape=jax.ShapeDtypeStruct(x_p.shape, jnp.float32),
    )(x_p, *mats, vecs)


def _pack_edge_mats(p):
    e = p["edge"]
    w1 = e[0]["W"].astype(jnp.float32)
    dm = _kron16(jnp.full((HID, HID), 1.0 / HID, jnp.float32))
    d1a = _kron16(w1[:HID])
    d1b = _kron16(w1[HID:])
    d2 = _kron16(e[1]["W"])
    d3 = _kron16(e[2]["W"])
    w4 = jnp.pad(e[3]["W"].astype(jnp.float32), ((0, 0), (0, HID - 1)))
    d4 = _kron16(w4)
    bmat = jnp.zeros((HID, HID), jnp.float32).at[0].set(1.0)
    db = _kron16(bmat)
    dc = jnp.kron(jnp.eye(PACK, dtype=jnp.float32),
                  jnp.zeros((HID, 1), jnp.float32).at[0, 0].set(1.0))
    b4row = jnp.full((1, LANES), e[3]["b"][0], jnp.float32)
    vecs = jnp.concatenate([
        _tile16(e[0]["b"]), _tile16(e[0]["g"]), _tile16(e[0]["beta"]),
        _tile16(e[1]["b"]), _tile16(e[1]["g"]), _tile16(e[1]["beta"]),
        _tile16(e[2]["b"]), _tile16(e[2]["g"]), _tile16(e[2]["beta"]),
        b4row,
        jnp.zeros((6, LANES), jnp.float32)], axis=0)
    return dm, d1a, d1b, d2, d3, d4, db, dc, vecs


def _pack_node_mats(p):
    n = p["node"]
    w1 = n[0]["W"].astype(jnp.float32)
    dm = _kron16(jnp.full((HID, HID), 1.0 / HID, jnp.float32))
    da = _kron16(w1[:HID])
    dbm = _kron16(w1[HID:2 * HID])
    dc = _kron16(w1[2 * HID:])
    d2 = _kron16(n[1]["W"])
    d3 = _kron16(n[2]["W"])
    d4 = _kron16(n[3]["W"])
    vecs = jnp.concatenate([
        _tile16(n[0]["b"]), _tile16(n[0]["g"]), _tile16(n[0]["beta"]),
        _tile16(n[1]["b"]), _tile16(n[1]["g"]), _tile16(n[1]["beta"]),
        _tile16(n[2]["b"]), _tile16(n[2]["g"]), _tile16(n[2]["beta"]),
        _tile16(n[3]["b"]), _tile16(n[3]["g"]), _tile16(n[3]["beta"]),
        jnp.zeros((4, LANES), jnp.float32)], axis=0)
    return dm, da, dbm, dc, d2, d3, d4, vecs


def _pack_inp_mats(p):
    pi = p["inp"]
    win = pi["W"].astype(jnp.float32)
    wpad = jnp.zeros((HID, HID), jnp.float32).at[:win.shape[0]].set(win)
    dm = _kron16(jnp.full((HID, HID), 1.0 / HID, jnp.float32))
    d1 = _kron16(wpad)
    vecs = jnp.concatenate([
        _tile16(pi["b"]), _tile16(pi["g"]), _tile16(pi["beta"]),
        jnp.zeros((5, LANES), jnp.float32)], axis=0)
    return dm, d1, vecs


def kernel(x, edge_index, params):
    n_nodes = x.shape[0]
    n_edges = edge_index.shape[1]
    ei = edge_index.astype(jnp.int32)
    start, end = ei[0], ei[1]

    emats_all = _pack_edge_mats(params)
    edm = emats_all[:7]
    edm_final = emats_all[:6] + (emats_all[7],)
    evecs = emats_all[8]
    nmats = _pack_node_mats(params)
    nvecs = nmats[7]
    imats = _pack_inp_mats(params)

    node_rows = n_nodes * HID // LANES
    edge_rows = n_edges * HID // LANES

    x_pad = jnp.pad(x.astype(jnp.float32), ((0, 0), (0, HID - x.shape[1])))
    x_p = _inp_pass(x_pad.reshape(node_rows, LANES), imats[:2], imats[2])

    for _ in range(3):
        xc = x_p.reshape(n_nodes, HID)
        xs_p = xc[start].reshape(edge_rows, LANES)
        xe_p = xc[end].reshape(edge_rows, LANES)
        ws_p, we_p = _edge_pass(xs_p, xe_p, edm, evecs, final=False)
        mi = jax.ops.segment_sum(ws_p.reshape(n_edges, HID), end,
                                 num_segments=n_nodes)
        mo = jax.ops.segment_sum(we_p.reshape(n_edges, HID), start,
                                 num_segments=n_nodes)
        x_p = _node_pass(mi.reshape(node_rows, LANES),
                         mo.reshape(node_rows, LANES),
                         x_p, nmats[:7], nvecs)

    xc = x_p.reshape(n_nodes, HID)
    xs_p = xc[start].reshape(edge_rows, LANES)
    xe_p = xc[end].reshape(edge_rows, LANES)
    out_p = _edge_pass(xs_p, xe_p, edm_final, evecs, final=True)
    return out_p.reshape(n_edges)


# speedup vs baseline: 33.3605x; 33.2636x over previous
import functools
import math

import numpy as np
import jax
import jax.numpy as jnp
from jax import lax
from jax.experimental import pallas as pl
from jax.experimental.pallas import tpu as pltpu
from jax.experimental.pallas import tpu_sc as plsc

HID = 8
NC = 2
NS = 16
NW = NC * NS
CHUNK = 128
KROWS = 8
PACK = 16
LANES = PACK * HID


def _pick_rows_blk(rows):
    for cand in (2048, 2000, 1600, 1280, 1024, 1000, 800, 640, 512, 400,
                 320, 256, 200, 160, 128, 80, 64, 40, 32, 16, 8):
        if rows % cand == 0:
            return cand
    return rows


def _kron16(w):
    return jnp.kron(jnp.eye(PACK, dtype=jnp.float32), w.astype(jnp.float32))


def _tile16(v):
    return jnp.tile(v.astype(jnp.float32).reshape(-1), PACK).reshape(1, LANES)


def _ln_tanh(h, dm, g, b):
    m = jnp.dot(h, dm)
    c = h - m
    v = jnp.dot(c * c, dm)
    return jnp.tanh(c * lax.rsqrt(v + 1e-5) * g + b)


def _edge_body(xs_ref, xe_ref, dm_ref, d1a_ref, d1b_ref, d2_ref, d3_ref,
               d4_ref, db_ref, v_ref, ws_ref, we_ref):
    xs = xs_ref[...]
    xe = xe_ref[...]
    dm = dm_ref[...]
    v = v_ref[...]
    h = jnp.dot(xs, d1a_ref[...]) + jnp.dot(xe, d1b_ref[...]) + v[0:1]
    h = _ln_tanh(h, dm, v[1:2], v[2:3])
    h = jnp.dot(h, d2_ref[...]) + v[3:4]
    h = _ln_tanh(h, dm, v[4:5], v[5:6])
    h = jnp.dot(h, d3_ref[...]) + v[6:7]
    h = _ln_tanh(h, dm, v[7:8], v[8:9])
    lg = jnp.dot(h, d4_ref[...]) + v[9:10]
    e = jnp.dot(jax.nn.sigmoid(lg), db_ref[...])
    ws_ref[...] = e * xs
    we_ref[...] = e * xe


def _edge_final_body(xs_ref, xe_ref, dm_ref, d1a_ref, d1b_ref, d2_ref,
                     d3_ref, d4_ref, dc_ref, v_ref, out_ref):
    xs = xs_ref[...]
    xe = xe_ref[...]
    dm = dm_ref[...]
    v = v_ref[...]
    h = jnp.dot(xs, d1a_ref[...]) + jnp.dot(xe, d1b_ref[...]) + v[0:1]
    h = _ln_tanh(h, dm, v[1:2], v[2:3])
    h = jnp.dot(h, d2_ref[...]) + v[3:4]
    h = _ln_tanh(h, dm, v[4:5], v[5:6])
    h = jnp.dot(h, d3_ref[...]) + v[6:7]
    h = _ln_tanh(h, dm, v[7:8], v[8:9])
    lg = jnp.dot(h, d4_ref[...]) + v[9:10]
    out_ref[...] = jnp.dot(lg, dc_ref[...])


def _node_body(acc_ref, x_ref, dm_ref, da_ref, dbm_ref, dc_ref,
               d2_ref, d3_ref, d4_ref, v_ref, out_ref):
    x = x_ref[...]
    dm = dm_ref[...]
    v = v_ref[...]
    nr = x_ref.shape[0]
    a = acc_ref[...]
    mi = a[0:nr] + a[2 * nr:3 * nr]
    mo = a[nr:2 * nr] + a[3 * nr:4 * nr]
    h = (jnp.dot(mi, da_ref[...]) + jnp.dot(mo, dbm_ref[...])
         + jnp.dot(x, dc_ref[...]) + v[0:1])
    h = _ln_tanh(h, dm, v[1:2], v[2:3])
    h = jnp.dot(h, d2_ref[...]) + v[3:4]
    h = _ln_tanh(h, dm, v[4:5], v[5:6])
    h = jnp.dot(h, d3_ref[...]) + v[6:7]
    h = _ln_tanh(h, dm, v[7:8], v[8:9])
    h = jnp.dot(h, d4_ref[...]) + v[9:10]
    h = _ln_tanh(h, dm, v[10:11], v[11:12])
    out_ref[...] = h + x


def _inp_body(x_ref, dm_ref, d1_ref, v_ref, out_ref):
    dm = dm_ref[...]
    v = v_ref[...]
    h = jnp.dot(x_ref[...], d1_ref[...]) + v[0:1]
    out_ref[...] = _ln_tanh(h, dm, v[1:2], v[2:3])


def _rep_spec(shape):
    nd = len(shape)
    zero = np.int32(0)
    return pl.BlockSpec(shape, lambda *_: (zero,) * nd)


def _edge_pass(xs_p, xe_p, mats, vecs, final):
    rows = xs_p.shape[0]
    blk = _pick_rows_blk(rows)
    grid = rows // blk
    row_spec = pl.BlockSpec((blk, LANES), lambda i: (i, np.int32(0)))
    in_specs = [row_spec, row_spec] + [_rep_spec(m.shape) for m in mats] + \
               [_rep_spec(vecs.shape)]
    if final:
        out_specs = pl.BlockSpec((blk, PACK), lambda i: (i, np.int32(0)))
        out_shape = jax.ShapeDtypeStruct((rows, PACK), jnp.float32)
        body = _edge_final_body
    else:
        out_specs = (row_spec, row_spec)
        out_shape = (jax.ShapeDtypeStruct((rows, LANES), jnp.float32),
                     jax.ShapeDtypeStruct((rows, LANES), jnp.float32))
        body = _edge_body
    return pl.pallas_call(
        body, grid=(grid,), in_specs=in_specs, out_specs=out_specs,
        out_shape=out_shape)(xs_p, xe_p, *mats, vecs)


def _node_pass(acc_p, x_p, mats, vecs):
    specs = [_rep_spec(a.shape) for a in (acc_p, x_p)] + \
            [_rep_spec(m.shape) for m in mats] + [_rep_spec(vecs.shape)]
    return pl.pallas_call(
        _node_body, in_specs=specs, out_specs=_rep_spec(x_p.shape),
        out_shape=jax.ShapeDtypeStruct(x_p.shape, jnp.float32),
    )(acc_p, x_p, *mats, vecs)


def _inp_pass(x_p, mats, vecs):
    specs = [_rep_spec(x_p.shape)] + [_rep_spec(m.shape) for m in mats] + \
            [_rep_spec(vecs.shape)]
    return pl.pallas_call(
        _inp_body, in_specs=specs, out_specs=_rep_spec(x_p.shape),
        out_shape=jax.ShapeDtypeStruct(x_p.shape, jnp.float32),
    )(x_p, *mats, vecs)


def _sc_mesh():
    return plsc.VectorSubcoreMesh(core_axis_name="c", subcore_axis_name="s")


def _sc_gather(x, sidx2, eidx2):
    n_rows = sidx2.shape[0]
    n_edges = n_rows * CHUNK
    n_stages = n_rows // KROWS
    base_st, extra = divmod(n_stages, NW)

    @functools.partial(
        pl.kernel,
        out_type=(jax.ShapeDtypeStruct((n_edges, HID), jnp.float32),
                  jax.ShapeDtypeStruct((n_edges, HID), jnp.float32)),
        mesh=_sc_mesh(),
        compiler_params=pltpu.CompilerParams(use_tc_tiling_on_sc=False),
        scratch_types=[
            pltpu.VMEM((KROWS, CHUNK), jnp.int32),
            pltpu.VMEM((KROWS, CHUNK), jnp.int32),
            pltpu.VMEM((KROWS * CHUNK, HID), jnp.float32),
            pltpu.VMEM((KROWS * CHUNK, HID), jnp.float32),
            pltpu.SemaphoreType.DMA,
        ])
    def gat(x_hbm, si_hbm, ei_hbm, xs_hbm, xe_hbm, si_t, ei_t, xs_t, xe_t,
            sem):
        wid = lax.axis_index("c") * NS + lax.axis_index("s")
        nst = jnp.where(wid < jnp.int32(extra), jnp.int32(base_st + 1),
                        jnp.int32(base_st))

        def body(t, carry):
            st = wid + t * jnp.int32(NW)
            rb = st * jnp.int32(KROWS)
            eb = rb * jnp.int32(CHUNK)
            pltpu.sync_copy(si_hbm.at[pl.ds(rb, KROWS)], si_t)
            pltpu.sync_copy(ei_hbm.at[pl.ds(rb, KROWS)], ei_t)
            descs = []
            for j in range(KROWS):
                descs.append(pltpu.async_copy(
                    x_hbm.at[si_t.at[np.int32(j)]],
                    xs_t.at[pl.ds(np.int32(j * CHUNK), CHUNK)], sem))
                descs.append(pltpu.async_copy(
                    x_hbm.at[ei_t.at[np.int32(j)]],
                    xe_t.at[pl.ds(np.int32(j * CHUNK), CHUNK)], sem))
            for d in descs:
                d.wait()
            pltpu.sync_copy(xs_t, xs_hbm.at[pl.ds(eb, KROWS * CHUNK)])
            pltpu.sync_copy(xe_t, xe_hbm.at[pl.ds(eb, KROWS * CHUNK)])
            return carry

        lax.fori_loop(jnp.int32(0), nst, body, jnp.int32(0))

    return gat(x, sidx2, eidx2)


def _sc_scatter(ws, we, eidx2, sidx2, n_nodes, zeros_slice):
    n_rows = eidx2.shape[0]
    n_stages = n_rows // KROWS
    base_st, extra = divmod(n_stages, NW)
    nslice = n_nodes // NS

    @functools.partial(
        pl.kernel,
        out_type=jax.ShapeDtypeStruct((NC, 2, n_nodes, HID), jnp.float32),
        mesh=_sc_mesh(),
        compiler_params=pltpu.CompilerParams(use_tc_tiling_on_sc=False),
        scratch_types=[
            pltpu.VMEM((KROWS, CHUNK), jnp.int32),
            pltpu.VMEM((KROWS, CHUNK), jnp.int32),
            pltpu.VMEM((KROWS * CHUNK, HID), jnp.float32),
            pltpu.VMEM((KROWS * CHUNK, HID), jnp.float32),
            pltpu.VMEM_SHARED((n_nodes, HID), jnp.float32),
            pltpu.VMEM_SHARED((n_nodes, HID), jnp.float32),
            pltpu.SemaphoreType.DMA,
        ])
    def scat(ws_hbm, we_hbm, ei_hbm, si_hbm, z_hbm, out_hbm,
             ei_t, si_t, ws_t, we_t, mi_sh, mo_sh, sem):
        c = lax.axis_index("c")
        s = lax.axis_index("s")
        wid = c * NS + s
        sb = s * jnp.int32(nslice)
        pltpu.sync_copy(z_hbm, mi_sh.at[pl.ds(sb, nslice)])
        pltpu.sync_copy(z_hbm, mo_sh.at[pl.ds(sb, nslice)])
        plsc.subcore_barrier()
        nst = jnp.where(wid < jnp.int32(extra), jnp.int32(base_st + 1),
                        jnp.int32(base_st))

        def body(t, carry):
            st = wid + t * jnp.int32(NW)
            rb = st * jnp.int32(KROWS)
            eb = rb * jnp.int32(CHUNK)
            pltpu.sync_copy(ei_hbm.at[pl.ds(rb, KROWS)], ei_t)
            pltpu.sync_copy(si_hbm.at[pl.ds(rb, KROWS)], si_t)
            pltpu.sync_copy(ws_hbm.at[pl.ds(eb, KROWS * CHUNK)], ws_t)
            pltpu.sync_copy(we_hbm.at[pl.ds(eb, KROWS * CHUNK)], we_t)
            descs = []
            for j in range(KROWS):
                descs.append(pltpu.async_copy(
                    ws_t.at[pl.ds(np.int32(j * CHUNK), CHUNK)],
                    mi_sh.at[ei_t.at[np.int32(j)]], sem, add=True))
                descs.append(pltpu.async_copy(
                    we_t.at[pl.ds(np.int32(j * CHUNK), CHUNK)],
                    mo_sh.at[si_t.at[np.int32(j)]], sem, add=True))
            for d in descs:
                d.wait()
            return carry

        lax.fori_loop(jnp.int32(0), nst, body, jnp.int32(0))
        plsc.subcore_barrier()
        pltpu.sync_copy(mi_sh.at[pl.ds(sb, nslice)],
                        out_hbm.at[c, np.int32(0), pl.ds(sb, nslice)])
        pltpu.sync_copy(mo_sh.at[pl.ds(sb, nslice)],
                        out_hbm.at[c, np.int32(1), pl.ds(sb, nslice)])

    return scat(ws, we, eidx2, sidx2, zeros_slice)


def _pack_edge_mats(p):
    e = p["edge"]
    w1 = e[0]["W"].astype(jnp.float32)
    dm = _kron16(jnp.full((HID, HID), 1.0 / HID, jnp.float32))
    d1a = _kron16(w1[:HID])
    d1b = _kron16(w1[HID:])
    d2 = _kron16(e[1]["W"])
    d3 = _kron16(e[2]["W"])
    w4 = jnp.pad(e[3]["W"].astype(jnp.float32), ((0, 0), (0, HID - 1)))
    d4 = _kron16(w4)
    bmat = jnp.zeros((HID, HID), jnp.float32).at[0].set(1.0)
    db = _kron16(bmat)
    dc = jnp.kron(jnp.eye(PACK, dtype=jnp.float32),
                  jnp.zeros((HID, 1), jnp.float32).at[0, 0].set(1.0))
    b4row = jnp.full((1, LANES), e[3]["b"][0], jnp.float32)
    vecs = jnp.concatenate([
        _tile16(e[0]["b"]), _tile16(e[0]["g"]), _tile16(e[0]["beta"]),
        _tile16(e[1]["b"]), _tile16(e[1]["g"]), _tile16(e[1]["beta"]),
        _tile16(e[2]["b"]), _tile16(e[2]["g"]), _tile16(e[2]["beta"]),
        b4row,
        jnp.zeros((6, LANES), jnp.float32)], axis=0)
    return dm, d1a, d1b, d2, d3, d4, db, dc, vecs


def _pack_node_mats(p):
    n = p["node"]
    w1 = n[0]["W"].astype(jnp.float32)
    dm = _kron16(jnp.full((HID, HID), 1.0 / HID, jnp.float32))
    da = _kron16(w1[:HID])
    dbm = _kron16(w1[HID:2 * HID])
    dc = _kron16(w1[2 * HID:])
    d2 = _kron16(n[1]["W"])
    d3 = _kron16(n[2]["W"])
    d4 = _kron16(n[3]["W"])
    vecs = jnp.concatenate([
        _tile16(n[0]["b"]), _tile16(n[0]["g"]), _tile16(n[0]["beta"]),
        _tile16(n[1]["b"]), _tile16(n[1]["g"]), _tile16(n[1]["beta"]),
        _tile16(n[2]["b"]), _tile16(n[2]["g"]), _tile16(n[2]["beta"]),
        _tile16(n[3]["b"]), _tile16(n[3]["g"]), _tile16(n[3]["beta"]),
        jnp.zeros((4, LANES), jnp.float32)], axis=0)
    return dm, da, dbm, dc, d2, d3, d4, vecs


def _pack_inp_mats(p):
    pi = p["inp"]
    win = pi["W"].astype(jnp.float32)
    wpad = jnp.zeros((HID, HID), jnp.float32).at[:win.shape[0]].set(win)
    dm = _kron16(jnp.full((HID, HID), 1.0 / HID, jnp.float32))
    d1 = _kron16(wpad)
    vecs = jnp.concatenate([
        _tile16(pi["b"]), _tile16(pi["g"]), _tile16(pi["beta"]),
        jnp.zeros((5, LANES), jnp.float32)], axis=0)
    return dm, d1, vecs


def kernel(x, edge_index, params):
    n_nodes = x.shape[0]
    n_edges = edge_index.shape[1]
    ei = edge_index.astype(jnp.int32)
    start, end = ei[0], ei[1]

    emats_all = _pack_edge_mats(params)
    edm = emats_all[:7]
    edm_final = emats_all[:6] + (emats_all[7],)
    evecs = emats_all[8]
    nmats = _pack_node_mats(params)
    nvecs = nmats[7]
    imats = _pack_inp_mats(params)

    node_rows = n_nodes * HID // LANES
    edge_rows = n_edges * HID // LANES
    sidx2 = start.reshape(n_edges // CHUNK, CHUNK)
    eidx2 = end.reshape(n_edges // CHUNK, CHUNK)
    zeros_slice = jnp.zeros((n_nodes // NS, HID), jnp.float32)

    x_pad = jnp.pad(x.astype(jnp.float32), ((0, 0), (0, HID - x.shape[1])))
    x_p = _inp_pass(x_pad.reshape(node_rows, LANES), imats[:2], imats[2])

    for _ in range(3):
        xs, xe = _sc_gather(x_p.reshape(n_nodes, HID), sidx2, eidx2)
        ws_p, we_p = _edge_pass(xs.reshape(edge_rows, LANES),
                                xe.reshape(edge_rows, LANES),
                                edm, evecs, final=False)
        acc = _sc_scatter(ws_p.reshape(n_edges, HID),
                          we_p.reshape(n_edges, HID),
                          eidx2, sidx2, n_nodes, zeros_slice)
        x_p = _node_pass(acc.reshape(NC * 2 * node_rows, LANES),
                         x_p, nmats[:7], nvecs)

    xs, xe = _sc_gather(x_p.reshape(n_nodes, HID), sidx2, eidx2)
    out_p = _edge_pass(xs.reshape(edge_rows, LANES),
                       xe.reshape(edge_rows, LANES),
                       edm_final, evecs, final=True)
    return out_p.reshape(n_edges)
